# Initial kernel scaffold; baseline (speedup 1.0000x reference)
import jax
import jax.numpy as jnp
from jax.experimental import pallas as pl


def kernel(inp, grid):
    raise NotImplementedError("write your pallas kernel here")


# speedup vs baseline: 1.0245x; 1.0245x over previous
import functools

import jax
import jax.numpy as jnp
from jax import lax
from jax.experimental import pallas as pl
from jax.experimental.pallas import tpu as pltpu
from jax.experimental.pallas import tpu_sc as plsc

NC = 2
NS = 16
L = 16
NW = NC * NS

CH = 64


def _floor_f32(x):
    t = x.astype(jnp.int32)
    tf = t.astype(jnp.float32)
    ti = jnp.where(tf > x, t - 1, t)
    return ti, ti.astype(jnp.float32)


def _make_sc_kernel(N, C, H, W, Ho, Wo):
    P = N * Ho * Wo
    HW = H * W
    assert P % NW == 0
    ppw = P // NW
    assert ppw % CH == 0
    nchunk = ppw // CH
    cg = C // L

    mesh = plsc.VectorSubcoreMesh(
        core_axis_name="c", subcore_axis_name="s", num_cores=NC,
        num_subcores=NS)

    @functools.partial(
        pl.kernel,
        out_type=jax.ShapeDtypeStruct((P, C), jnp.float32),
        mesh=mesh,
        scratch_types=[
            pltpu.VMEM((CH,), jnp.float32),
            pltpu.VMEM((CH,), jnp.float32),
            pltpu.VMEM((CH,), jnp.int32),
            pltpu.VMEM((CH,), jnp.int32),
            pltpu.VMEM((CH,), jnp.int32),
            pltpu.VMEM((CH,), jnp.int32),
            pltpu.VMEM((CH,), jnp.float32),
            pltpu.VMEM((CH,), jnp.float32),
            pltpu.VMEM((CH,), jnp.float32),
            pltpu.VMEM((CH,), jnp.float32),
            pltpu.VMEM((CH, C), jnp.float32),
            pltpu.VMEM((CH, C), jnp.float32),
            pltpu.VMEM((CH, C), jnp.float32),
            pltpu.VMEM((CH, C), jnp.float32),
            pltpu.VMEM((CH, C), jnp.float32),
            pltpu.SemaphoreType.DMA,
        ],
        compiler_params=pltpu.CompilerParams(use_tc_tiling_on_sc=False),
    )
    def grid_sample_sc(table_hbm, gx_hbm, gy_hbm, out_hbm,
                       gx_v, gy_v, i00_v, i10_v, i01_v, i11_v,
                       ax0_v, ax1_v, ay0_v, ay1_v,
                       r00_v, r10_v, r01_v, r11_v, out_v, sem):
        wid = lax.axis_index("s") * NC + lax.axis_index("c")
        base_w = wid * ppw
        nbase = (base_w // (Ho * Wo)) * HW

        @pl.loop(0, nchunk)
        def _chunk(k):
            base = base_w + k * CH
            pltpu.sync_copy(gx_hbm.at[pl.ds(base, CH)], gx_v)
            pltpu.sync_copy(gy_hbm.at[pl.ds(base, CH)], gy_v)

            for g in range(CH // L):
                sl = pl.ds(g * L, L)
                gx = gx_v[sl]
                gy = gy_v[sl]
                ix = (gx + 1.0) * (W * 0.5) - 0.5
                iy = (gy + 1.0) * (H * 0.5) - 0.5
                ix0i, ix0f = _floor_f32(ix)
                iy0i, iy0f = _floor_f32(iy)
                wx1 = ix - ix0f
                wy1 = iy - iy0f

                mx0 = (ix0f >= 0.0) & (ix0f <= W - 1.0)
                mx1 = (ix0f >= -1.0) & (ix0f <= W - 2.0)
                my0 = (iy0f >= 0.0) & (iy0f <= H - 1.0)
                my1 = (iy0f >= -1.0) & (iy0f <= H - 2.0)
                ax0_v[sl] = jnp.where(mx0, 1.0 - wx1, 0.0)
                ax1_v[sl] = jnp.where(mx1, wx1, 0.0)
                ay0_v[sl] = jnp.where(my0, 1.0 - wy1, 0.0)
                ay1_v[sl] = jnp.where(my1, wy1, 0.0)

                xi0 = jnp.clip(ix0i, 0, W - 1)
                xi1 = jnp.clip(ix0i + 1, 0, W - 1)
                yi0 = jnp.clip(iy0i, 0, H - 1) * W + nbase
                yi1 = jnp.clip(iy0i + 1, 0, H - 1) * W + nbase
                i00_v[sl] = yi0 + xi0
                i10_v[sl] = yi0 + xi1
                i01_v[sl] = yi1 + xi0
                i11_v[sl] = yi1 + xi1

            c00 = pltpu.async_copy(table_hbm.at[i00_v], r00_v, sem)
            c10 = pltpu.async_copy(table_hbm.at[i10_v], r10_v, sem)
            c01 = pltpu.async_copy(table_hbm.at[i01_v], r01_v, sem)
            c11 = pltpu.async_copy(table_hbm.at[i11_v], r11_v, sem)
            c00.wait()
            c10.wait()
            c01.wait()
            c11.wait()

            @pl.loop(0, CH // L)
            def _grp(g):
                sl = pl.ds(g * L, L)
                a0 = ax0_v[sl]
                a1 = ax1_v[sl]
                b0 = ay0_v[sl]
                b1 = ay1_v[sl]
                for ll in range(L):
                    i = g * L + ll
                    lane = jnp.full((L,), ll, jnp.int32)
                    a0s = a0.at[lane].get(mode="promise_in_bounds")
                    a1s = a1.at[lane].get(mode="promise_in_bounds")
                    b0s = b0.at[lane].get(mode="promise_in_bounds")
                    b1s = b1.at[lane].get(mode="promise_in_bounds")
                    for j in range(cg):
                        cs = pl.ds(j * L, L)
                        t0 = a0s * r00_v[i, cs] + a1s * r10_v[i, cs]
                        t1 = a0s * r01_v[i, cs] + a1s * r11_v[i, cs]
                        out_v[i, cs] = b0s * t0 + b1s * t1

            pltpu.sync_copy(out_v, out_hbm.at[pl.ds(base, CH)])

    return grid_sample_sc


def kernel(inp, grid):
    N, C, H, W = inp.shape
    _, Ho, Wo, _ = grid.shape
    table = inp.transpose(0, 2, 3, 1).reshape(N * H * W, C)
    gx = grid[..., 0].reshape(-1)
    gy = grid[..., 1].reshape(-1)
    sc = _make_sc_kernel(N, C, H, W, Ho, Wo)
    out_rows = sc(table, gx, gy)
    return out_rows.reshape(N, Ho, Wo, C).transpose(0, 3, 1, 2)


# speedup vs baseline: 1.3982x; 1.3648x over previous
import functools

import jax
import jax.numpy as jnp
from jax import lax
from jax.experimental import pallas as pl
from jax.experimental.pallas import tpu as pltpu
from jax.experimental.pallas import tpu_sc as plsc

NC = 2
NS = 16
L = 16
NW = NC * NS

CH = 64


def _floor_f32(x):
    t = x.astype(jnp.int32)
    tf = t.astype(jnp.float32)
    ti = jnp.where(tf > x, t - 1, t)
    return ti, ti.astype(jnp.float32)


def _make_sc_kernel(N, C, H, W, Ho, Wo):
    P = N * Ho * Wo
    HW = H * W
    assert P % NW == 0
    ppw = P // NW
    assert ppw % CH == 0
    nchunk = ppw // CH
    assert nchunk % 2 == 0
    cg = C // L

    mesh = plsc.VectorSubcoreMesh(
        core_axis_name="c", subcore_axis_name="s", num_cores=NC,
        num_subcores=NS)

    @functools.partial(
        pl.kernel,
        out_type=jax.ShapeDtypeStruct((P, C), jnp.float32),
        mesh=mesh,
        scratch_types=[
            pltpu.VMEM((ppw,), jnp.float32),
            pltpu.VMEM((ppw,), jnp.float32),
            [pltpu.VMEM((4, CH), jnp.int32) for _ in range(2)],
            [pltpu.VMEM((4, CH), jnp.float32) for _ in range(2)],
            [[pltpu.VMEM((CH, C), jnp.float32) for _ in range(4)]
             for _ in range(2)],
            pltpu.VMEM((CH, C), jnp.float32),
            [pltpu.SemaphoreType.DMA for _ in range(2)],
        ],
        compiler_params=pltpu.CompilerParams(use_tc_tiling_on_sc=False),
    )
    def grid_sample_sc(gx_hbm, gy_hbm, table_hbm, out_hbm,
                       gx_v, gy_v, idx_v, w_v, rows_v, out_v, sems):
        wid = lax.axis_index("s") * NC + lax.axis_index("c")
        base_w = wid * ppw
        nbase = (base_w // (Ho * Wo)) * HW

        pltpu.sync_copy(gx_hbm.at[pl.ds(base_w, ppw)], gx_v)
        pltpu.sync_copy(gy_hbm.at[pl.ds(base_w, ppw)], gy_v)

        def stage_chunk(c, b):
            for g in range(CH // L):
                sl = pl.ds(c * CH + g * L, L)
                osl = pl.ds(g * L, L)
                gx = gx_v[sl]
                gy = gy_v[sl]
                ix = (gx + 1.0) * (W * 0.5) - 0.5
                iy = (gy + 1.0) * (H * 0.5) - 0.5
                ix0i, ix0f = _floor_f32(ix)
                iy0i, iy0f = _floor_f32(iy)
                wx1 = ix - ix0f
                wy1 = iy - iy0f

                mx0 = (ix0f >= 0.0) & (ix0f <= W - 1.0)
                mx1 = (ix0f >= -1.0) & (ix0f <= W - 2.0)
                my0 = (iy0f >= 0.0) & (iy0f <= H - 1.0)
                my1 = (iy0f >= -1.0) & (iy0f <= H - 2.0)
                w_v[b][0, osl] = jnp.where(mx0, 1.0 - wx1, 0.0)
                w_v[b][1, osl] = jnp.where(mx1, wx1, 0.0)
                w_v[b][2, osl] = jnp.where(my0, 1.0 - wy1, 0.0)
                w_v[b][3, osl] = jnp.where(my1, wy1, 0.0)

                xi0 = jnp.clip(ix0i, 0, W - 1)
                xi1 = jnp.clip(ix0i + 1, 0, W - 1)
                yi0 = jnp.clip(iy0i, 0, H - 1) * W + nbase
                yi1 = jnp.clip(iy0i + 1, 0, H - 1) * W + nbase
                idx_v[b][0, osl] = yi0 + xi0
                idx_v[b][1, osl] = yi0 + xi1
                idx_v[b][2, osl] = yi1 + xi0
                idx_v[b][3, osl] = yi1 + xi1

            for q in range(4):
                pltpu.async_copy(table_hbm.at[idx_v[b].at[q]], rows_v[b][q],
                                 sems[b])

        def finish_chunk(c, b):
            for q in range(4):
                pltpu.make_async_copy(table_hbm.at[idx_v[b].at[q]],
                                      rows_v[b][q], sems[b]).wait()

            @pl.loop(0, CH // L)
            def _grp(g):
                sl = pl.ds(g * L, L)
                a0 = w_v[b][0, sl]
                a1 = w_v[b][1, sl]
                b0 = w_v[b][2, sl]
                b1 = w_v[b][3, sl]
                for ll in range(L):
                    i = g * L + ll
                    lane = jnp.full((L,), ll, jnp.int32)
                    a0s = a0.at[lane].get(mode="promise_in_bounds")
                    a1s = a1.at[lane].get(mode="promise_in_bounds")
                    b0s = b0.at[lane].get(mode="promise_in_bounds")
                    b1s = b1.at[lane].get(mode="promise_in_bounds")
                    for j in range(cg):
                        cs = pl.ds(j * L, L)
                        t0 = (a0s * rows_v[b][0][i, cs]
                              + a1s * rows_v[b][1][i, cs])
                        t1 = (a0s * rows_v[b][2][i, cs]
                              + a1s * rows_v[b][3][i, cs])
                        out_v[i, cs] = b0s * t0 + b1s * t1

            pltpu.sync_copy(out_v, out_hbm.at[pl.ds(base_w + c * CH, CH)])

        stage_chunk(0, 0)

        @pl.loop(0, nchunk, step=2)
        def _chunk(k):
            for b in range(2):
                c = k + b

                @pl.when(c + 1 < nchunk)
                def _prefetch():
                    stage_chunk(c + 1, (b + 1) % 2)

                finish_chunk(c, b)

    return grid_sample_sc


def kernel(inp, grid):
    N, C, H, W = inp.shape
    _, Ho, Wo, _ = grid.shape
    table = inp.transpose(0, 2, 3, 1).reshape(N * H * W, C)
    gx = grid[..., 0].reshape(-1)
    gy = grid[..., 1].reshape(-1)
    sc = _make_sc_kernel(N, C, H, W, Ho, Wo)
    out_rows = sc(gx, gy, table)
    return out_rows.reshape(N, Ho, Wo, C).transpose(0, 3, 1, 2)
